# baseline (speedup 1.0000x reference)
import functools

import jax
import jax.numpy as jnp
from jax import lax
from jax.experimental import pallas as pl
from jax.experimental.pallas import tpu as pltpu
from jax.experimental.pallas import tpu_sc as plsc

_NC = 2
_NS = 16
_NW = _NC * _NS
_L = 16
_CH = 8


@jax.jit
def _sc_lookup(weight, idx_flat):
    n, = idx_flat.shape
    V, D = weight.shape
    b_per_w = n // _NW
    nchunks = b_per_w // _CH
    nper = nchunks // 4
    assert nper * 4 == nchunks
    mesh = plsc.VectorSubcoreMesh(core_axis_name="c", subcore_axis_name="s")

    @functools.partial(
        pl.kernel,
        mesh=mesh,
        out_type=jax.ShapeDtypeStruct((n, D), jnp.float32),
        scratch_types=(
            [pltpu.VMEM((V, D), jnp.float32),
             pltpu.VMEM((b_per_w,), jnp.int32)]
            + [pltpu.VMEM((_CH, D), jnp.float32)] * 2
            + [pltpu.SemaphoreType.DMA] * 3
        ),
    )
    def k(table_hbm, idx_hbm, out_hbm, table_v, idx_v, b0, b1,
          gsem, w0, w1):
        bufs = (b0, b1)
        wsems = (w0, w1)
        sid = lax.axis_index("s")
        wid = sid * _NC + lax.axis_index("c")
        base = wid * b_per_w

        pltpu.sync_copy(table_hbm, table_v)
        pltpu.sync_copy(idx_hbm.at[pl.ds(base, b_per_w)], idx_v)

        def start_gather(j):
            pltpu.async_copy(
                table_hbm.at[idx_v.at[pl.ds(j * _CH, _CH)]], bufs[0], gsem)

        def wait_gather():
            pltpu.make_async_copy(
                out_hbm.at[pl.ds(base, _CH)], bufs[0], gsem).wait()

        def start_write(j, b):
            pltpu.async_copy(
                bufs[b], out_hbm.at[pl.ds(base + j * _CH, _CH)], wsems[b])

        def wait_write(b):
            pltpu.make_async_copy(
                bufs[b], out_hbm.at[pl.ds(base, _CH)], wsems[b]).wait()

        def fill(ivec, h, b):
            for r in range(_CH):
                i = ivec[h * _CH + r]

                def dgroup(d, carry):
                    c0 = d * 16 * _L
                    vals = [table_v[i, pl.ds(c0 + u * _L, _L)]
                            for u in range(16)]
                    for u in range(16):
                        bufs[b][r, pl.ds(c0 + u * _L, _L)] = vals[u]
                    return carry

                lax.fori_loop(0, D // (16 * _L), dgroup, 0)

        start_gather(0)

        def body(t, carry):
            j0 = t * 4
            row0 = j0 * _CH
            ivec1 = idx_v[pl.ds(row0 + _CH, 2 * _CH)]
            ivec2 = idx_v[pl.ds(row0 + 2 * _CH, 2 * _CH)]

            wait_gather()
            start_write(j0, 0)

            @pl.when(t > 0)
            def _():
                wait_write(1)
            fill(ivec1, 0, 1)
            start_write(j0 + 1, 1)

            wait_write(0)
            fill(ivec1, 1, 0)
            start_write(j0 + 2, 0)

            wait_write(1)
            fill(ivec2, 1, 1)
            start_write(j0 + 3, 1)

            wait_write(0)

            @pl.when(t + 1 < nper)
            def _():
                start_gather(j0 + 4)
            return carry

        lax.fori_loop(0, nper, body, 0)
        wait_write(1)

    return k(weight, idx_flat)


def kernel(indices, weight):
    B, S = indices.shape
    D = weight.shape[1]
    idx_flat = indices.reshape(-1).astype(jnp.int32)
    out = _sc_lookup(weight, idx_flat)
    return out.reshape(B, S, D)


# speedup vs baseline: 1.3765x; 1.3765x over previous
import functools

import jax
import jax.numpy as jnp
from jax import lax
from jax.experimental import pallas as pl
from jax.experimental.pallas import tpu as pltpu
from jax.experimental.pallas import tpu_sc as plsc

_NC = 2
_NS = 16
_NW = _NC * _NS
_K = 16


@functools.partial(jax.jit, static_argnames=("chunk",))
def _sc_lookup(weight, idx_flat, chunk):
    n, = idx_flat.shape
    V, D = weight.shape
    b_per_w = n // _NW
    nchunks = b_per_w // chunk
    assert chunk % 8 == 0
    mesh = plsc.VectorSubcoreMesh(core_axis_name="c", subcore_axis_name="s")

    @functools.partial(
        pl.kernel,
        mesh=mesh,
        out_type=jax.ShapeDtypeStruct((n, D), jnp.float32),
        scratch_types=(
            [pltpu.VMEM((b_per_w,), jnp.int32)]
            + [pltpu.VMEM((chunk, D), jnp.float32)] * 3
            + [pltpu.SemaphoreType.DMA] * 6
        ),
    )
    def k(table_hbm, idx_hbm, out_hbm, idx_v, *rest):
        bufs = rest[:3]
        gsems = rest[3:6]
        wsems = rest[6:9]
        sid = lax.axis_index("s")
        wid = sid * _NC + lax.axis_index("c")
        base = wid * b_per_w

        pltpu.sync_copy(idx_hbm.at[pl.ds(base, b_per_w)], idx_v)

        def start_gather(j, b):
            pltpu.async_copy(
                table_hbm.at[idx_v.at[pl.ds(j * chunk, chunk)]],
                bufs[b], gsems[b])

        def start_write(j, b):
            pltpu.async_copy(
                bufs[b], out_hbm.at[pl.ds(base + j * chunk, chunk)], wsems[b])

        def wait_gather(b):
            pltpu.make_async_copy(
                out_hbm.at[pl.ds(base, chunk)], bufs[b], gsems[b]).wait()

        def wait_write(b):
            pltpu.make_async_copy(
                bufs[b], out_hbm.at[pl.ds(base, chunk)], wsems[b]).wait()

        for b in range(3):
            start_gather(b, b)

        def body(jj, carry):
            for b in range(3):
                j = jj * 3 + b
                wait_gather(b)
                start_write(j, b)
                wait_write(b)

                @pl.when(j + 3 < nchunks)
                def _():
                    start_gather(j + 3, b)
            return carry

        lax.fori_loop(0, nchunks // 3, body, 0)
        for j in range((nchunks // 3) * 3, nchunks):
            b = j % 3
            wait_gather(b)
            start_write(j, b)
            wait_write(b)

    return k(weight, idx_flat)


def kernel(indices, weight):
    B, S = indices.shape
    V, D = weight.shape
    idx_flat = indices.reshape(-1).astype(jnp.int32)
    w_big = jnp.tile(weight, (_K, 1))
    idx_spread = idx_flat + (jnp.arange(idx_flat.shape[0], dtype=jnp.int32)
                             % _K) * V
    out = _sc_lookup(w_big, idx_spread, chunk=40)
    return out.reshape(B, S, D)
